# Initial kernel scaffold; baseline (speedup 1.0000x reference)
import jax
import jax.numpy as jnp
from jax.experimental import pallas as pl


def kernel(layer_input, gate_weight, gate_up_W, down_W):
    raise NotImplementedError("write your pallas kernel here")


# speedup vs baseline: 2.5847x; 2.5847x over previous
import functools

import jax
import jax.numpy as jnp
from jax.experimental import pallas as pl
from jax.experimental.pallas import tpu as pltpu

E = 64
TOP_K = 2
D_MODEL = 1024
D_FF = 512


def _moe_dense_kernel(x_ref, gw_ref, gup_ref, dw_ref, out_ref,
                      i1_ref, i2_ref, w1_ref, w2_ref):
    e = pl.program_id(0)

    @pl.when(e == 0)
    def _init():
        x = x_ref[...]
        logits = jax.lax.dot_general(
            x, gw_ref[...], (((1,), (1,)), ((), ())),
            preferred_element_type=jnp.float32)
        m = jnp.max(logits, axis=1, keepdims=True)
        p = jnp.exp(logits - m)
        probs = p / jnp.sum(p, axis=1, keepdims=True)
        iota = jax.lax.broadcasted_iota(jnp.int32, probs.shape, 1)
        m1 = jnp.max(probs, axis=1, keepdims=True)
        i1 = jnp.min(jnp.where(probs == m1, iota, E), axis=1, keepdims=True)
        probs2 = jnp.where(iota == i1, -1.0, probs)
        m2 = jnp.max(probs2, axis=1, keepdims=True)
        i2 = jnp.min(jnp.where(probs2 == m2, iota, E), axis=1, keepdims=True)
        denom = m1 + m2
        i1_ref[...] = i1
        i2_ref[...] = i2
        w1_ref[...] = m1 / denom
        w2_ref[...] = m2 / denom
        out_ref[...] = jnp.zeros_like(out_ref)

    x = x_ref[...]
    h = jax.lax.dot_general(
        x, gup_ref[0], (((1,), (0,)), ((), ())),
        preferred_element_type=jnp.float32)
    gate_h = h[:, :D_FF]
    up_h = h[:, D_FF:]
    act = gate_h * jax.lax.logistic(gate_h) * up_h
    y = jax.lax.dot_general(
        act, dw_ref[0], (((1,), (0,)), ((), ())),
        preferred_element_type=jnp.float32)
    coef = (jnp.where(i1_ref[...] == e, w1_ref[...], 0.0)
            + jnp.where(i2_ref[...] == e, w2_ref[...], 0.0))
    out_ref[...] += coef * y


@jax.jit
def _moe(x, gate_weight, gate_up_W, down_W):
    S = x.shape[0]
    out = pl.pallas_call(
        _moe_dense_kernel,
        grid=(E,),
        in_specs=[
            pl.BlockSpec((S, D_MODEL), lambda e: (0, 0)),
            pl.BlockSpec((E, D_MODEL), lambda e: (0, 0)),
            pl.BlockSpec((1, D_MODEL, 2 * D_FF), lambda e: (e, 0, 0)),
            pl.BlockSpec((1, D_FF, D_MODEL), lambda e: (e, 0, 0)),
        ],
        out_specs=pl.BlockSpec((S, D_MODEL), lambda e: (0, 0)),
        out_shape=jax.ShapeDtypeStruct((S, D_MODEL), jnp.float32),
        scratch_shapes=[
            pltpu.VMEM((S, 1), jnp.int32),
            pltpu.VMEM((S, 1), jnp.int32),
            pltpu.VMEM((S, 1), jnp.float32),
            pltpu.VMEM((S, 1), jnp.float32),
        ],
    )(x, gate_weight, gate_up_W, down_W)
    return out


def kernel(layer_input, gate_weight, gate_up_W, down_W):
    B, S, H = layer_input.shape
    x = layer_input.reshape(-1, H)
    out = _moe(x, gate_weight, gate_up_W, down_W)
    return out.reshape(B, S, H)


# speedup vs baseline: 3.8554x; 1.4916x over previous
import functools

import jax
import jax.numpy as jnp
from jax.experimental import pallas as pl
from jax.experimental.pallas import tpu as pltpu

E = 64
TOP_K = 2
D_MODEL = 1024
D_FF = 512
S = 2048
P = S * TOP_K
T = 128

NT = (E - 1) + -(-(P - (E - 1)) // T)
PAD = NT * T


def _router_kernel(x_ref, gw_ref, idx_ref, w_ref):
    logits = jax.lax.dot_general(
        gw_ref[...], x_ref[...], (((1,), (1,)), ((), ())),
        preferred_element_type=jnp.float32)
    iota = jax.lax.broadcasted_iota(jnp.int32, logits.shape, 0)
    m1 = jnp.max(logits, axis=0, keepdims=True)
    i1 = jnp.min(jnp.where(logits == m1, iota, E), axis=0, keepdims=True)
    l2 = jnp.where(iota == i1, -jnp.inf, logits)
    m2 = jnp.max(l2, axis=0, keepdims=True)
    i2 = jnp.min(jnp.where(l2 == m2, iota, E), axis=0, keepdims=True)
    w1 = 1.0 / (1.0 + jnp.exp(m2 - m1))
    idx_ref[...] = jnp.concatenate([i1, i2], axis=0)
    w_ref[...] = jnp.concatenate([w1, 1.0 - w1], axis=0)


def _grouped_kernel(te_ref, nvalid_ref, stok_ref, scoef_ref,
                    x_ref, gup_ref, dw_ref, out_ref, g_ref, y_ref):
    i = pl.program_id(0)

    @pl.when(i == 0)
    def _init():
        out_ref[...] = jnp.zeros_like(out_ref)

    n = nvalid_ref[i]
    base = i * T

    def gather_body(j, _):
        tok = stok_ref[base + j]
        g_ref[pl.ds(j, 1), :] = x_ref[pl.ds(tok, 1), :]
        return 0

    jax.lax.fori_loop(0, n, gather_body, 0)

    h = jax.lax.dot_general(
        g_ref[...], gup_ref[0], (((1,), (0,)), ((), ())),
        preferred_element_type=jnp.float32)
    gate_h = h[:, :D_FF]
    up_h = h[:, D_FF:]
    act = gate_h * jax.lax.logistic(gate_h) * up_h
    y_ref[...] = jax.lax.dot_general(
        act, dw_ref[0], (((1,), (0,)), ((), ())),
        preferred_element_type=jnp.float32)

    def scatter_body(j, _):
        tok = stok_ref[base + j]
        c = scoef_ref[base + j]
        out_ref[pl.ds(tok, 1), :] += c * y_ref[pl.ds(j, 1), :]
        return 0

    jax.lax.fori_loop(0, n, scatter_body, 0)


@jax.jit
def _moe(x, gate_weight, gate_up_W, down_W):
    idx, w = pl.pallas_call(
        _router_kernel,
        in_specs=[
            pl.BlockSpec((S, D_MODEL), lambda: (0, 0)),
            pl.BlockSpec((E, D_MODEL), lambda: (0, 0)),
        ],
        out_specs=[
            pl.BlockSpec((TOP_K, S), lambda: (0, 0)),
            pl.BlockSpec((TOP_K, S), lambda: (0, 0)),
        ],
        out_shape=[
            jax.ShapeDtypeStruct((TOP_K, S), jnp.int32),
            jax.ShapeDtypeStruct((TOP_K, S), jnp.float32),
        ],
    )(x, gate_weight)

    e_pair = idx.T.reshape(P)
    c_pair = w.T.reshape(P)
    t_pair = (jnp.arange(P, dtype=jnp.int32) // TOP_K)

    onehot = (e_pair[:, None] == jnp.arange(E, dtype=jnp.int32)[None, :])
    counts = jnp.sum(onehot.astype(jnp.int32), axis=0)
    rank = jnp.take_along_axis(
        jnp.cumsum(onehot.astype(jnp.int32), axis=0), e_pair[:, None], axis=1
    )[:, 0] - 1
    tiles_per_e = -(-counts // T)
    tile_start = jnp.concatenate(
        [jnp.zeros((1,), jnp.int32), jnp.cumsum(tiles_per_e)[:-1]])
    slot = tile_start[e_pair] * T + rank

    slot_token = jnp.zeros((PAD,), jnp.int32).at[slot].set(t_pair)
    slot_coef = jnp.zeros((PAD,), jnp.float32).at[slot].set(c_pair)

    tile_ids = jnp.arange(NT, dtype=jnp.int32)
    tile_end = tile_start + tiles_per_e
    in_e = ((tile_ids[:, None] >= tile_start[None, :])
            & (tile_ids[:, None] < tile_end[None, :]))
    tile_expert = jnp.where(
        jnp.any(in_e, axis=1),
        jnp.argmax(in_e, axis=1).astype(jnp.int32), E - 1)
    nvalid = jnp.where(
        jnp.any(in_e, axis=1),
        jnp.clip(counts[tile_expert] - (tile_ids - tile_start[tile_expert]) * T,
                 0, T), 0).astype(jnp.int32)

    grid_spec = pltpu.PrefetchScalarGridSpec(
        num_scalar_prefetch=4,
        grid=(NT,),
        in_specs=[
            pl.BlockSpec((S, D_MODEL), lambda i, te, nv, st, sc: (0, 0)),
            pl.BlockSpec((1, D_MODEL, 2 * D_FF),
                         lambda i, te, nv, st, sc: (te[i], 0, 0)),
            pl.BlockSpec((1, D_FF, D_MODEL),
                         lambda i, te, nv, st, sc: (te[i], 0, 0)),
        ],
        out_specs=pl.BlockSpec((S, D_MODEL), lambda i, te, nv, st, sc: (0, 0)),
        scratch_shapes=[
            pltpu.VMEM((T, D_MODEL), jnp.float32),
            pltpu.VMEM((T, D_MODEL), jnp.float32),
        ],
    )
    out = pl.pallas_call(
        _grouped_kernel,
        grid_spec=grid_spec,
        out_shape=jax.ShapeDtypeStruct((S, D_MODEL), jnp.float32),
    )(tile_expert, nvalid, slot_token, slot_coef, x, gate_up_W, down_W)
    return out


def kernel(layer_input, gate_weight, gate_up_W, down_W):
    B, Sb, H = layer_input.shape
    x = layer_input.reshape(-1, H)
    out = _moe(x, gate_weight, gate_up_W, down_W)
    return out.reshape(B, Sb, H)
